# baseline (speedup 1.0000x reference)
import functools

import jax
import jax.numpy as jnp
from jax import lax
from jax.experimental import pallas as pl
from jax.experimental.pallas import tpu as pltpu
from jax.experimental.pallas import tpu_sc as plsc

IGNORE_INDEX = -100
CONFIDENCE = 0.9

_LANES = 16
_CHUNK = 128


def _make_sc_gather(b, v, nc, ns):
    nw = nc * ns
    bpw = b // nw
    vrows = v // _CHUNK
    mesh = plsc.VectorSubcoreMesh(core_axis_name="c", subcore_axis_name="s")

    @functools.partial(
        pl.kernel,
        mesh=mesh,
        out_type=jax.ShapeDtypeStruct((b, _CHUNK), jnp.float32),
        scratch_types=[
            pltpu.VMEM((bpw,), jnp.int32),
            pltpu.VMEM((bpw,), jnp.int32),
            pltpu.VMEM((bpw, _CHUNK), jnp.float32),
            pltpu.SemaphoreType.DMA,
        ],
    )
    def gather_kernel(x_hbm, t_hbm, out_hbm, t_v, row_v, chunk_v, sem):
        wid = lax.axis_index("s") * nc + lax.axis_index("c")
        base = wid * bpw
        pltpu.sync_copy(t_hbm.at[pl.ds(base, bpw)], t_v)
        for j in range(bpw // _LANES):
            tv = jnp.maximum(t_v[pl.ds(j * _LANES, _LANES)], 0)
            iv = lax.iota(jnp.int32, _LANES) + (base + j * _LANES)
            row_v[pl.ds(j * _LANES, _LANES)] = iv * vrows + (tv >> 7)
        pltpu.async_copy(x_hbm.at[row_v], chunk_v, sem).wait()
        pltpu.sync_copy(chunk_v, out_hbm.at[pl.ds(base, bpw)])

    return gather_kernel


def _loss_body(x_ref, t_ref, xt16_ref, oh_ref, out_ref):
    x = x_ref[...]
    t = t_ref[0, 0, :]
    xt16 = xt16_ref[0]

    br, v = x.shape

    lane = t & (_CHUNK - 1)
    col = jax.lax.broadcasted_iota(jnp.int32, (br, _CHUNK), 1)
    xt = jnp.sum(jnp.where(col == lane[:, None], xt16, 0.0), axis=1)
    zero_col = v + IGNORE_INDEX

    sv = oh_ref[0, 0]
    log_sv = jnp.log(sv)
    sum_oh = sv * (v - 1)
    c_ent = sv * log_sv * (v - 1)

    nchunk = 10
    c = v // nchunk
    m_acc = jnp.max(x_ref[:, : c], axis=1, keepdims=True)
    r_acc = jnp.sum(x_ref[:, : c], axis=1)
    for k in range(1, nchunk):
        xk = x_ref[:, k * c:(k + 1) * c]
        m_acc = jnp.maximum(m_acc, jnp.max(xk, axis=1, keepdims=True))
        r_acc = r_acc + jnp.sum(xk, axis=1)
    m = m_acc
    s = jnp.sum(jnp.exp(x - m), axis=1)
    lse = m[:, 0] + jnp.log(s)

    d = sv * (r_acc - x[:, zero_col])

    lp_t = xt - lse
    row = (c_ent - d + lse * sum_oh
           - jnp.where(t != zero_col, sv * (log_sv - lp_t), 0.0)
           + CONFIDENCE * (jnp.log(CONFIDENCE) - lp_t))
    row = jnp.where(t != IGNORE_INDEX, row, 0.0)
    out_ref[...] = jnp.sum(row).reshape(1, 1, 1)


@jax.jit
def kernel(output, target, one_hot):
    b, v = output.shape
    info = plsc.get_sparse_core_info()
    xt16 = _make_sc_gather(b, v, info.num_cores, info.num_subcores)(
        output.reshape(b * v // _CHUNK, _CHUNK), target)

    br = 128
    nb = b // br
    target3 = target.reshape(nb, 1, br)
    xt16_3 = xt16.reshape(nb, br, _CHUNK)

    partials = pl.pallas_call(
        _loss_body,
        grid=(nb,),
        in_specs=[
            pl.BlockSpec((br, v), lambda i: (i, 0)),
            pl.BlockSpec((1, 1, br), lambda i: (i, 0, 0)),
            pl.BlockSpec((1, br, _CHUNK), lambda i: (i, 0, 0)),
            pl.BlockSpec((1, v), lambda i: (0, 0)),
        ],
        out_specs=pl.BlockSpec((1, 1, 1), lambda i: (i, 0, 0)),
        out_shape=jax.ShapeDtypeStruct((nb, 1, 1), jnp.float32),
    )(output, target3, xt16_3, one_hot)

    return jnp.sum(partials) / b


# speedup vs baseline: 2.8785x; 2.8785x over previous
import functools

import jax
import jax.numpy as jnp
from jax import lax
from jax.experimental import pallas as pl
from jax.experimental.pallas import tpu as pltpu
from jax.experimental.pallas import tpu_sc as plsc

IGNORE_INDEX = -100
CONFIDENCE = 0.9

_LANES = 16
_CHUNK = 128


def _loss_body(x_ref, t_ref, oh_ref, rowval_ref, chunk_ref):
    x = x_ref[...]
    t = t_ref[0, 0, :]

    br, v = x.shape
    zero_col = v + IGNORE_INDEX

    sv = oh_ref[0, 0]
    log_sv = jnp.log(sv)
    sum_oh = sv * (v - 1)
    c_ent = sv * log_sv * (v - 1)

    nchunk = 10
    c = v // nchunk
    m_acc = jnp.max(x_ref[:, :c], axis=1, keepdims=True)
    r_acc = jnp.sum(x_ref[:, :c], axis=1)
    for k in range(1, nchunk):
        xk = x_ref[:, k * c:(k + 1) * c]
        m_acc = jnp.maximum(m_acc, jnp.max(xk, axis=1, keepdims=True))
        r_acc = r_acc + jnp.sum(xk, axis=1)
    m = m_acc

    s = jnp.sum(jnp.exp(x - m), axis=1)
    lse = m[:, 0] + jnp.log(s)

    d = sv * (r_acc - x[:, zero_col])

    for r in range(br):
        start = pl.multiple_of((t_ref[0, 0, r] >> 7) * _CHUNK, _CHUNK)
        chunk_ref[r, :] = x_ref[r, pl.ds(start, _CHUNK)]
    lane = t & (_CHUNK - 1)
    col = lax.broadcasted_iota(jnp.int32, (br, _CHUNK), 1)
    xt = jnp.sum(jnp.where(col == lane[:, None], chunk_ref[...], 0.0), axis=1)

    lp = xt - lse
    row = (c_ent - d + lse * sum_oh
           - jnp.where(t != zero_col, sv * (log_sv - lp), 0.0)
           + CONFIDENCE * (jnp.log(CONFIDENCE) - lp))
    rowval_ref[...] = jnp.where(t != IGNORE_INDEX, row, 0.0).reshape(1, 1, br)


def _make_sc_reduce(b, nc, ns):
    nw = nc * ns
    bpw = b // nw
    mesh = plsc.VectorSubcoreMesh(core_axis_name="c", subcore_axis_name="s")

    @functools.partial(
        pl.kernel,
        mesh=mesh,
        out_type=jax.ShapeDtypeStruct((nw, _LANES), jnp.float32),
        scratch_types=[
            pltpu.VMEM((bpw,), jnp.float32),
            pltpu.VMEM((_LANES,), jnp.float32),
        ],
    )
    def reduce_kernel(rowval_hbm, out_hbm, rowval_v, acc_v):
        wid = lax.axis_index("s") * nc + lax.axis_index("c")
        base = wid * bpw
        pltpu.sync_copy(rowval_hbm.at[pl.ds(base, bpw)], rowval_v)
        acc = jnp.zeros((_LANES,), jnp.float32)
        for j in range(bpw // _LANES):
            acc = acc + rowval_v[pl.ds(j * _LANES, _LANES)]
        acc_v[...] = acc
        pltpu.sync_copy(acc_v, out_hbm.at[wid])

    return reduce_kernel


@jax.jit
def kernel(output, target, one_hot):
    b, v = output.shape
    br = 128
    nb = b // br
    target3 = target.reshape(nb, 1, br)

    rowvals = pl.pallas_call(
        _loss_body,
        grid=(nb,),
        in_specs=[
            pl.BlockSpec((br, v), lambda i: (i, 0)),
            pl.BlockSpec((1, 1, br), lambda i: (i, 0, 0)),
            pl.BlockSpec((1, v), lambda i: (0, 0)),
        ],
        out_specs=pl.BlockSpec((1, 1, br), lambda i: (i, 0, 0)),
        out_shape=jax.ShapeDtypeStruct((nb, 1, br), jnp.float32),
        scratch_shapes=[pltpu.VMEM((br, _CHUNK), jnp.float32)],
    )(output, target3, one_hot)

    info = plsc.get_sparse_core_info()
    partials = _make_sc_reduce(b, info.num_cores, info.num_subcores)(
        rowvals.reshape(b))

    return jnp.sum(partials) / b
